# baseline (device time: 22673 ns/iter reference)
import jax
import jax.numpy as jnp
from jax import lax
from jax.experimental import pallas as pl
from jax.experimental.pallas import tpu as pltpu

N_DEV = 8
B = 2
SQ = 128
SKV = 1024
HQ = 4
DH = 64
D_MODEL = 512
D_QK = HQ * DH
KV_PER = SKV // N_DEV
RQ = SQ // N_DEV
NEG = -1e9


def kernel(x, Wq, K_ext, V_ext, Wo):
    k_t = jnp.transpose(K_ext, (0, 2, 3, 1))
    v_t = jnp.transpose(V_ext, (0, 2, 3, 1))

    def body(x_hbm, wq_hbm, k_hbm, v_hbm, wo_hbm, out_ref,
             xv, wqv, kv, vv, wov,
             upart, spart, ucomb, scomb, omine,
             in_sems, osem,
             usend, urecv, ssend, srecv, osend, orecv):
        my = lax.axis_index("i")

        fetch = [
            pltpu.make_async_copy(x_hbm, xv, in_sems.at[0]),
            pltpu.make_async_copy(wq_hbm, wqv, in_sems.at[1]),
            pltpu.make_async_copy(k_hbm, kv, in_sems.at[2]),
            pltpu.make_async_copy(v_hbm, vv, in_sems.at[3]),
            pltpu.make_async_copy(wo_hbm, wov, in_sems.at[4]),
        ]
        for f in fetch:
            f.start()

        barrier_sem = pltpu.get_barrier_semaphore()
        for nbr in range(N_DEV):
            @pl.when(nbr != my)
            def _():
                pl.semaphore_signal(
                    barrier_sem, inc=1,
                    device_id=(nbr,), device_id_type=pl.DeviceIdType.MESH,
                )
        pl.semaphore_wait(barrier_sem, N_DEV - 1)

        for f in fetch[:4]:
            f.wait()

        qi = lax.broadcasted_iota(jnp.int32, (SQ, KV_PER), 0)
        kj = lax.broadcasted_iota(jnp.int32, (SQ, KV_PER), 1) + my * KV_PER
        mask = (jnp.abs(qi - kj) <= 128) | (kj < 32) | (qi < 32)

        for b in range(B):
            q_b = jnp.dot(xv[b], wqv[...],
                          preferred_element_type=jnp.float32)
            ms, ls = [], []
            for hh in range(HQ):
                sl = slice(hh * DH, (hh + 1) * DH)
                scores = lax.dot_general(
                    q_b[:, sl], kv[b, hh],
                    (((1,), (0,)), ((), ())),
                    preferred_element_type=jnp.float32,
                )
                s = jnp.where(mask, scores * 0.125, NEG)
                m = jnp.max(s, axis=1, keepdims=True)
                p = jnp.exp(s - m)
                l = jnp.sum(p, axis=1, keepdims=True)
                u = lax.dot_general(
                    p, vv[b, hh],
                    (((1,), (1,)), ((), ())),
                    preferred_element_type=jnp.float32,
                )
                for d in range(N_DEV):
                    upart[d, b, :, sl] = u[d * RQ:(d + 1) * RQ]
                ms.append(m)
                ls.append(l)
            st = jnp.concatenate(ms + ls, axis=1)
            for d in range(N_DEV):
                spart[d, b] = st[d * RQ:(d + 1) * RQ]

        for peer in range(N_DEV):
            @pl.when(peer != my)
            def _():
                pltpu.make_async_remote_copy(
                    src_ref=upart.at[peer], dst_ref=ucomb.at[my],
                    send_sem=usend.at[peer], recv_sem=urecv.at[my],
                    device_id=(peer,), device_id_type=pl.DeviceIdType.MESH,
                ).start()
                pltpu.make_async_remote_copy(
                    src_ref=spart.at[peer], dst_ref=scomb.at[my],
                    send_sem=ssend.at[peer], recv_sem=srecv.at[my],
                    device_id=(peer,), device_id_type=pl.DeviceIdType.MESH,
                ).start()
        ucomb[pl.ds(my, 1)] = upart[my][None]
        scomb[pl.ds(my, 1)] = spart[my][None]

        for o in range(N_DEV):
            @pl.when(o != my)
            def _():
                pltpu.make_async_remote_copy(
                    src_ref=ucomb.at[o], dst_ref=ucomb.at[o],
                    send_sem=usend.at[o], recv_sem=urecv.at[o],
                    device_id=(my,), device_id_type=pl.DeviceIdType.MESH,
                ).wait_recv()
                pltpu.make_async_remote_copy(
                    src_ref=scomb.at[o], dst_ref=scomb.at[o],
                    send_sem=ssend.at[o], recv_sem=srecv.at[o],
                    device_id=(my,), device_id_type=pl.DeviceIdType.MESH,
                ).wait_recv()

        fetch[4].wait()

        for b in range(B):
            ctx_heads = []
            for hh in range(HQ):
                sl = slice(hh * DH, (hh + 1) * DH)
                ms = [scomb[o, b][:, hh:hh + 1] for o in range(N_DEV)]
                ls = [scomb[o, b][:, HQ + hh:HQ + hh + 1] for o in range(N_DEV)]
                M = ms[0]
                for o in range(1, N_DEV):
                    M = jnp.maximum(M, ms[o])
                ws = [jnp.exp(ms[o] - M) for o in range(N_DEV)]
                L = ws[0] * ls[0]
                acc = ws[0] * ucomb[0, b][:, sl]
                for o in range(1, N_DEV):
                    L = L + ws[o] * ls[o]
                    acc = acc + ws[o] * ucomb[o, b][:, sl]
                ctx_heads.append(acc / L)
            ctx = jnp.concatenate(ctx_heads, axis=1)
            omine[b] = jnp.dot(ctx, wov[...],
                               preferred_element_type=jnp.float32)

        own = pltpu.make_async_copy(
            omine, out_ref.at[:, pl.ds(my * RQ, RQ)], osem)
        own.start()

        for peer in range(N_DEV):
            @pl.when(peer != my)
            def _():
                pltpu.make_async_remote_copy(
                    src_ref=omine, dst_ref=out_ref.at[:, pl.ds(my * RQ, RQ)],
                    send_sem=osend.at[peer], recv_sem=orecv.at[my],
                    device_id=(peer,), device_id_type=pl.DeviceIdType.MESH,
                ).start()
        own.wait()
        for o in range(N_DEV):
            @pl.when(o != my)
            def _():
                pltpu.make_async_remote_copy(
                    src_ref=omine, dst_ref=out_ref.at[:, pl.ds(o * RQ, RQ)],
                    send_sem=osend.at[o], recv_sem=orecv.at[o],
                    device_id=(my,), device_id_type=pl.DeviceIdType.MESH,
                ).wait_recv()

        for peer in range(N_DEV):
            @pl.when(peer != my)
            def _():
                pltpu.make_async_remote_copy(
                    src_ref=upart.at[peer], dst_ref=ucomb.at[my],
                    send_sem=usend.at[peer], recv_sem=urecv.at[my],
                    device_id=(peer,), device_id_type=pl.DeviceIdType.MESH,
                ).wait_send()
                pltpu.make_async_remote_copy(
                    src_ref=spart.at[peer], dst_ref=scomb.at[my],
                    send_sem=ssend.at[peer], recv_sem=srecv.at[my],
                    device_id=(peer,), device_id_type=pl.DeviceIdType.MESH,
                ).wait_send()
                pltpu.make_async_remote_copy(
                    src_ref=omine, dst_ref=out_ref.at[:, pl.ds(my * RQ, RQ)],
                    send_sem=osend.at[peer], recv_sem=orecv.at[my],
                    device_id=(peer,), device_id_type=pl.DeviceIdType.MESH,
                ).wait_send()

    return pl.pallas_call(
        body,
        out_shape=jax.ShapeDtypeStruct((B, SQ, D_MODEL), jnp.float32),
        in_specs=[pl.BlockSpec(memory_space=pltpu.MemorySpace.HBM)] * 5,
        out_specs=pl.BlockSpec(memory_space=pltpu.MemorySpace.HBM),
        scratch_shapes=[
            pltpu.VMEM((B, SQ, D_MODEL), jnp.float32),
            pltpu.VMEM((D_MODEL, D_QK), jnp.float32),
            pltpu.VMEM((B, HQ, DH, KV_PER), jnp.float32),
            pltpu.VMEM((B, HQ, DH, KV_PER), jnp.float32),
            pltpu.VMEM((D_QK, D_MODEL), jnp.float32),
            pltpu.VMEM((N_DEV, B, RQ, D_QK), jnp.float32),
            pltpu.VMEM((N_DEV, B, RQ, 2 * HQ), jnp.float32),
            pltpu.VMEM((N_DEV, B, RQ, D_QK), jnp.float32),
            pltpu.VMEM((N_DEV, B, RQ, 2 * HQ), jnp.float32),
            pltpu.VMEM((B, RQ, D_MODEL), jnp.float32),
            pltpu.SemaphoreType.DMA((5,)),
            pltpu.SemaphoreType.DMA,
            pltpu.SemaphoreType.DMA((N_DEV,)),
            pltpu.SemaphoreType.DMA((N_DEV,)),
            pltpu.SemaphoreType.DMA((N_DEV,)),
            pltpu.SemaphoreType.DMA((N_DEV,)),
            pltpu.SemaphoreType.DMA((N_DEV,)),
            pltpu.SemaphoreType.DMA((N_DEV,)),
        ],
        compiler_params=pltpu.CompilerParams(collective_id=0),
    )(x, Wq, k_t, v_t, Wo)


# device time: 22540 ns/iter; 1.0059x vs baseline; 1.0059x over previous
import jax
import jax.numpy as jnp
from jax import lax
from jax.experimental import pallas as pl
from jax.experimental.pallas import tpu as pltpu

N_DEV = 8
B = 2
SQ = 128
SKV = 1024
HQ = 4
DH = 64
D_MODEL = 512
D_QK = HQ * DH
KV_PER = SKV // N_DEV
RQ = SQ // N_DEV
NEG = -1e9


def kernel(x, Wq, K_ext, V_ext, Wo):
    k_t = jnp.transpose(K_ext, (0, 2, 3, 1))
    v_t = jnp.transpose(V_ext, (0, 2, 3, 1))

    def body(xv, wqv, k_hbm, v_hbm, wov, out_ref,
             kv, vv,
             upart, spart, ucomb, scomb, omine,
             in_sems, osem,
             usend, urecv, ssend, srecv, osend, orecv):
        my = lax.axis_index("i")

        fetch = [
            pltpu.make_async_copy(k_hbm, kv, in_sems.at[0]),
            pltpu.make_async_copy(v_hbm, vv, in_sems.at[1]),
        ]
        for f in fetch:
            f.start()

        barrier_sem = pltpu.get_barrier_semaphore()
        for nbr in range(N_DEV):
            @pl.when(nbr != my)
            def _():
                pl.semaphore_signal(
                    barrier_sem, inc=1,
                    device_id=(nbr,), device_id_type=pl.DeviceIdType.MESH,
                )
        pl.semaphore_wait(barrier_sem, N_DEV - 1)

        for f in fetch:
            f.wait()

        qi = lax.broadcasted_iota(jnp.int32, (SQ, KV_PER), 0)
        kj = lax.broadcasted_iota(jnp.int32, (SQ, KV_PER), 1) + my * KV_PER
        mask = (jnp.abs(qi - kj) <= 128) | (kj < 32) | (qi < 32)

        for b in range(B):
            q_b = jnp.dot(xv[b], wqv[...],
                          preferred_element_type=jnp.float32)
            ms, ls = [], []
            for hh in range(HQ):
                sl = slice(hh * DH, (hh + 1) * DH)
                scores = lax.dot_general(
                    q_b[:, sl], kv[b, hh],
                    (((1,), (0,)), ((), ())),
                    preferred_element_type=jnp.float32,
                )
                s = jnp.where(mask, scores * 0.125, NEG)
                m = jnp.max(s, axis=1, keepdims=True)
                p = jnp.exp(s - m)
                l = jnp.sum(p, axis=1, keepdims=True)
                u = lax.dot_general(
                    p, vv[b, hh],
                    (((1,), (1,)), ((), ())),
                    preferred_element_type=jnp.float32,
                )
                for d in range(N_DEV):
                    upart[d, b, :, sl] = u[d * RQ:(d + 1) * RQ]
                ms.append(m)
                ls.append(l)
            st = jnp.concatenate(ms + ls, axis=1)
            for d in range(N_DEV):
                spart[d, b] = st[d * RQ:(d + 1) * RQ]

        for peer in range(N_DEV):
            @pl.when(peer != my)
            def _():
                pltpu.make_async_remote_copy(
                    src_ref=upart.at[peer], dst_ref=ucomb.at[my],
                    send_sem=usend.at[peer], recv_sem=urecv.at[my],
                    device_id=(peer,), device_id_type=pl.DeviceIdType.MESH,
                ).start()
                pltpu.make_async_remote_copy(
                    src_ref=spart.at[peer], dst_ref=scomb.at[my],
                    send_sem=ssend.at[peer], recv_sem=srecv.at[my],
                    device_id=(peer,), device_id_type=pl.DeviceIdType.MESH,
                ).start()
        ucomb[pl.ds(my, 1)] = upart[my][None]
        scomb[pl.ds(my, 1)] = spart[my][None]

        for o in range(N_DEV):
            @pl.when(o != my)
            def _():
                pltpu.make_async_remote_copy(
                    src_ref=ucomb.at[o], dst_ref=ucomb.at[o],
                    send_sem=usend.at[o], recv_sem=urecv.at[o],
                    device_id=(my,), device_id_type=pl.DeviceIdType.MESH,
                ).wait_recv()
                pltpu.make_async_remote_copy(
                    src_ref=scomb.at[o], dst_ref=scomb.at[o],
                    send_sem=ssend.at[o], recv_sem=srecv.at[o],
                    device_id=(my,), device_id_type=pl.DeviceIdType.MESH,
                ).wait_recv()

        for b in range(B):
            ctx_heads = []
            for hh in range(HQ):
                sl = slice(hh * DH, (hh + 1) * DH)
                ms = [scomb[o, b][:, hh:hh + 1] for o in range(N_DEV)]
                ls = [scomb[o, b][:, HQ + hh:HQ + hh + 1] for o in range(N_DEV)]
                M = ms[0]
                for o in range(1, N_DEV):
                    M = jnp.maximum(M, ms[o])
                ws = [jnp.exp(ms[o] - M) for o in range(N_DEV)]
                L = ws[0] * ls[0]
                acc = ws[0] * ucomb[0, b][:, sl]
                for o in range(1, N_DEV):
                    L = L + ws[o] * ls[o]
                    acc = acc + ws[o] * ucomb[o, b][:, sl]
                ctx_heads.append(acc / L)
            ctx = jnp.concatenate(ctx_heads, axis=1)
            omine[b] = jnp.dot(ctx, wov[...],
                               preferred_element_type=jnp.float32)

        own = pltpu.make_async_copy(
            omine, out_ref.at[:, pl.ds(my * RQ, RQ)], osem)
        own.start()

        for peer in range(N_DEV):
            @pl.when(peer != my)
            def _():
                pltpu.make_async_remote_copy(
                    src_ref=omine, dst_ref=out_ref.at[:, pl.ds(my * RQ, RQ)],
                    send_sem=osend.at[peer], recv_sem=orecv.at[my],
                    device_id=(peer,), device_id_type=pl.DeviceIdType.MESH,
                ).start()
        own.wait()
        for o in range(N_DEV):
            @pl.when(o != my)
            def _():
                pltpu.make_async_remote_copy(
                    src_ref=omine, dst_ref=out_ref.at[:, pl.ds(o * RQ, RQ)],
                    send_sem=osend.at[o], recv_sem=orecv.at[o],
                    device_id=(my,), device_id_type=pl.DeviceIdType.MESH,
                ).wait_recv()

        for peer in range(N_DEV):
            @pl.when(peer != my)
            def _():
                pltpu.make_async_remote_copy(
                    src_ref=upart.at[peer], dst_ref=ucomb.at[my],
                    send_sem=usend.at[peer], recv_sem=urecv.at[my],
                    device_id=(peer,), device_id_type=pl.DeviceIdType.MESH,
                ).wait_send()
                pltpu.make_async_remote_copy(
                    src_ref=spart.at[peer], dst_ref=scomb.at[my],
                    send_sem=ssend.at[peer], recv_sem=srecv.at[my],
                    device_id=(peer,), device_id_type=pl.DeviceIdType.MESH,
                ).wait_send()
                pltpu.make_async_remote_copy(
                    src_ref=omine, dst_ref=out_ref.at[:, pl.ds(my * RQ, RQ)],
                    send_sem=osend.at[peer], recv_sem=orecv.at[my],
                    device_id=(peer,), device_id_type=pl.DeviceIdType.MESH,
                ).wait_send()

    return pl.pallas_call(
        body,
        out_shape=jax.ShapeDtypeStruct((B, SQ, D_MODEL), jnp.float32),
        in_specs=[
            pl.BlockSpec(memory_space=pltpu.VMEM),
            pl.BlockSpec(memory_space=pltpu.VMEM),
            pl.BlockSpec(memory_space=pltpu.MemorySpace.HBM),
            pl.BlockSpec(memory_space=pltpu.MemorySpace.HBM),
            pl.BlockSpec(memory_space=pltpu.VMEM),
        ],
        out_specs=pl.BlockSpec(memory_space=pltpu.MemorySpace.HBM),
        scratch_shapes=[
            pltpu.VMEM((B, HQ, DH, KV_PER), jnp.float32),
            pltpu.VMEM((B, HQ, DH, KV_PER), jnp.float32),
            pltpu.VMEM((N_DEV, B, RQ, D_QK), jnp.float32),
            pltpu.VMEM((N_DEV, B, RQ, 2 * HQ), jnp.float32),
            pltpu.VMEM((N_DEV, B, RQ, D_QK), jnp.float32),
            pltpu.VMEM((N_DEV, B, RQ, 2 * HQ), jnp.float32),
            pltpu.VMEM((B, RQ, D_MODEL), jnp.float32),
            pltpu.SemaphoreType.DMA((2,)),
            pltpu.SemaphoreType.DMA,
            pltpu.SemaphoreType.DMA((N_DEV,)),
            pltpu.SemaphoreType.DMA((N_DEV,)),
            pltpu.SemaphoreType.DMA((N_DEV,)),
            pltpu.SemaphoreType.DMA((N_DEV,)),
            pltpu.SemaphoreType.DMA((N_DEV,)),
            pltpu.SemaphoreType.DMA((N_DEV,)),
        ],
        compiler_params=pltpu.CompilerParams(collective_id=0),
    )(x, Wq, k_t, v_t, Wo)


# device time: 17457 ns/iter; 1.2988x vs baseline; 1.2912x over previous
import jax
import jax.numpy as jnp
from jax import lax
from jax.experimental import pallas as pl
from jax.experimental.pallas import tpu as pltpu

N_DEV = 8
B = 2
SQ = 128
SKV = 1024
HQ = 4
DH = 64
D_MODEL = 512
D_QK = HQ * DH
KV_PER = SKV // N_DEV
RQ = SQ // N_DEV
NEG = -1e9


def kernel(x, Wq, K_ext, V_ext, Wo):
    k_shard = K_ext.reshape(B, KV_PER, D_QK)
    v_shard = V_ext.reshape(B, KV_PER, D_QK)

    def body(x_ref, wq_ref, k_ref, v_ref, wo_ref, out_ref,
             upart, spart, ucomb, scomb, omine,
             usend, urecv, ssend, srecv, osend, orecv):
        my = lax.axis_index("i")

        barrier_sem = pltpu.get_barrier_semaphore()
        for nbr in range(N_DEV):
            @pl.when(nbr != my)
            def _():
                pl.semaphore_signal(
                    barrier_sem, inc=1,
                    device_id=(nbr,), device_id_type=pl.DeviceIdType.MESH,
                )
        pl.semaphore_wait(barrier_sem, N_DEV - 1)

        qi = lax.broadcasted_iota(jnp.int32, (SQ, KV_PER), 0)
        kj = lax.broadcasted_iota(jnp.int32, (SQ, KV_PER), 1) + my * KV_PER
        mask = (jnp.abs(qi - kj) <= 128) | (kj < 32) | (qi < 32)

        for b in range(B):
            q_b = jnp.dot(x_ref[b], wq_ref[...],
                          preferred_element_type=jnp.float32)
            ms, ls = [], []
            for hh in range(HQ):
                sl = slice(hh * DH, (hh + 1) * DH)
                scores = lax.dot_general(
                    q_b[:, sl], k_ref[b][:, sl],
                    (((1,), (1,)), ((), ())),
                    preferred_element_type=jnp.float32,
                )
                s = jnp.where(mask, scores * 0.125, NEG)
                m = jnp.max(s, axis=1, keepdims=True)
                p = jnp.exp(s - m)
                l = jnp.sum(p, axis=1, keepdims=True)
                u = jnp.dot(p, v_ref[b][:, sl],
                            preferred_element_type=jnp.float32)
                for d in range(N_DEV):
                    upart[d, b, :, sl] = u[d * RQ:(d + 1) * RQ]
                ms.append(m)
                ls.append(l)
            st = jnp.concatenate(ms + ls, axis=1)
            for d in range(N_DEV):
                spart[d, b] = st[d * RQ:(d + 1) * RQ]

            for peer in range(N_DEV):
                @pl.when(peer != my)
                def _():
                    pltpu.make_async_remote_copy(
                        src_ref=upart.at[peer, b], dst_ref=ucomb.at[my, b],
                        send_sem=usend.at[peer, b], recv_sem=urecv.at[my, b],
                        device_id=(peer,), device_id_type=pl.DeviceIdType.MESH,
                    ).start()
                    pltpu.make_async_remote_copy(
                        src_ref=spart.at[peer, b], dst_ref=scomb.at[my, b],
                        send_sem=ssend.at[peer, b], recv_sem=srecv.at[my, b],
                        device_id=(peer,), device_id_type=pl.DeviceIdType.MESH,
                    ).start()
            ucomb[pl.ds(my, 1), b] = upart[my, b][None]
            scomb[pl.ds(my, 1), b] = spart[my, b][None]

        for b in range(B):
            for o in range(N_DEV):
                @pl.when(o != my)
                def _():
                    pltpu.make_async_remote_copy(
                        src_ref=ucomb.at[o, b], dst_ref=ucomb.at[o, b],
                        send_sem=usend.at[o, b], recv_sem=urecv.at[o, b],
                        device_id=(my,), device_id_type=pl.DeviceIdType.MESH,
                    ).wait_recv()
                    pltpu.make_async_remote_copy(
                        src_ref=scomb.at[o, b], dst_ref=scomb.at[o, b],
                        send_sem=ssend.at[o, b], recv_sem=srecv.at[o, b],
                        device_id=(my,), device_id_type=pl.DeviceIdType.MESH,
                    ).wait_recv()

            ctx_heads = []
            for hh in range(HQ):
                sl = slice(hh * DH, (hh + 1) * DH)
                ms = [scomb[o, b][:, hh:hh + 1] for o in range(N_DEV)]
                ls = [scomb[o, b][:, HQ + hh:HQ + hh + 1] for o in range(N_DEV)]
                M = ms[0]
                for o in range(1, N_DEV):
                    M = jnp.maximum(M, ms[o])
                ws = [jnp.exp(ms[o] - M) for o in range(N_DEV)]
                L = ws[0] * ls[0]
                acc = ws[0] * ucomb[0, b][:, sl]
                for o in range(1, N_DEV):
                    L = L + ws[o] * ls[o]
                    acc = acc + ws[o] * ucomb[o, b][:, sl]
                ctx_heads.append(acc / L)
            ctx = jnp.concatenate(ctx_heads, axis=1)
            o_b = jnp.dot(ctx, wo_ref[...],
                          preferred_element_type=jnp.float32)
            omine[b] = o_b
            out_ref[b, pl.ds(my * RQ, RQ), :] = o_b

            for peer in range(N_DEV):
                @pl.when(peer != my)
                def _():
                    pltpu.make_async_remote_copy(
                        src_ref=omine.at[b],
                        dst_ref=out_ref.at[b, pl.ds(my * RQ, RQ)],
                        send_sem=osend.at[peer, b], recv_sem=orecv.at[my, b],
                        device_id=(peer,), device_id_type=pl.DeviceIdType.MESH,
                    ).start()

        for b in range(B):
            for o in range(N_DEV):
                @pl.when(o != my)
                def _():
                    pltpu.make_async_remote_copy(
                        src_ref=omine.at[b],
                        dst_ref=out_ref.at[b, pl.ds(o * RQ, RQ)],
                        send_sem=osend.at[o, b], recv_sem=orecv.at[o, b],
                        device_id=(my,), device_id_type=pl.DeviceIdType.MESH,
                    ).wait_recv()

        for b in range(B):
            for peer in range(N_DEV):
                @pl.when(peer != my)
                def _():
                    pltpu.make_async_remote_copy(
                        src_ref=upart.at[peer, b], dst_ref=ucomb.at[my, b],
                        send_sem=usend.at[peer, b], recv_sem=urecv.at[my, b],
                        device_id=(peer,), device_id_type=pl.DeviceIdType.MESH,
                    ).wait_send()
                    pltpu.make_async_remote_copy(
                        src_ref=spart.at[peer, b], dst_ref=scomb.at[my, b],
                        send_sem=ssend.at[peer, b], recv_sem=srecv.at[my, b],
                        device_id=(peer,), device_id_type=pl.DeviceIdType.MESH,
                    ).wait_send()
                    pltpu.make_async_remote_copy(
                        src_ref=omine.at[b],
                        dst_ref=out_ref.at[b, pl.ds(my * RQ, RQ)],
                        send_sem=osend.at[peer, b], recv_sem=orecv.at[my, b],
                        device_id=(peer,), device_id_type=pl.DeviceIdType.MESH,
                    ).wait_send()

    return pl.pallas_call(
        body,
        out_shape=jax.ShapeDtypeStruct((B, SQ, D_MODEL), jnp.float32),
        in_specs=[pl.BlockSpec(memory_space=pltpu.VMEM)] * 5,
        out_specs=pl.BlockSpec(memory_space=pltpu.VMEM),
        scratch_shapes=[
            pltpu.VMEM((N_DEV, B, RQ, D_QK), jnp.float32),
            pltpu.VMEM((N_DEV, B, RQ, 2 * HQ), jnp.float32),
            pltpu.VMEM((N_DEV, B, RQ, D_QK), jnp.float32),
            pltpu.VMEM((N_DEV, B, RQ, 2 * HQ), jnp.float32),
            pltpu.VMEM((B, RQ, D_MODEL), jnp.float32),
            pltpu.SemaphoreType.DMA((N_DEV, B)),
            pltpu.SemaphoreType.DMA((N_DEV, B)),
            pltpu.SemaphoreType.DMA((N_DEV, B)),
            pltpu.SemaphoreType.DMA((N_DEV, B)),
            pltpu.SemaphoreType.DMA((N_DEV, B)),
            pltpu.SemaphoreType.DMA((N_DEV, B)),
        ],
        compiler_params=pltpu.CompilerParams(collective_id=0),
    )(x, Wq, k_shard, v_shard, Wo)
